# baseline (speedup 1.0000x reference)
import functools
import math

import jax
import jax.numpy as jnp
import numpy as np
from jax import lax
from jax.experimental import pallas as pl
from jax.experimental.pallas import tpu as pltpu
from jax.experimental.pallas import tpu_sc as plsc

D_MODEL = 128
MAXLEN = 512
B = 1024
L = 200

BL = B * L
NW = 32
CH = 128
ROWS_PER_W = BL // NW
NCH = ROWS_PER_W // CH
NB = 2
PE_EXT = L + CH
VREGS = D_MODEL // 16


def _sinusoidal_pe(max_len, d_model):
    pe = np.zeros((max_len, d_model), dtype=np.float32)
    position = np.arange(0, max_len, dtype=np.float32)[:, None]
    div_term = np.exp(
        np.arange(0, d_model, 2, dtype=np.float32) * -(math.log(10000.0) / d_model)
    )
    pe[:, 0::2] = np.sin(position * div_term)
    pe[:, 1::2] = np.cos(position * div_term)
    return pe


_PE = _sinusoidal_pe(MAXLEN, D_MODEL)[:L]
_PE_EXT = np.concatenate([_PE, _PE[: PE_EXT - L]], axis=0)


def _make_kernel():
    mesh = plsc.VectorSubcoreMesh(core_axis_name="c", subcore_axis_name="s")

    scratch = [pltpu.VMEM((NCH, 1, CH), jnp.int32),
               pltpu.VMEM((PE_EXT, D_MODEL), jnp.float32)]
    scratch += [pltpu.VMEM((CH, D_MODEL), jnp.float32) for _ in range(2 * NB)]
    scratch += [pltpu.SemaphoreType.DMA for _ in range(2 * NB)]

    @functools.partial(
        pl.kernel,
        mesh=mesh,
        out_type=jax.ShapeDtypeStruct((BL, D_MODEL), jnp.float32),
        scratch_types=scratch,
    )
    def emb_kernel(idx_hbm, table_hbm, pe_hbm, out_hbm, idx_v, pe_v, *bufs):
        gbuf = bufs[0:NB]
        sbuf = bufs[NB:2 * NB]
        gsem = bufs[2 * NB:3 * NB]
        ssem = bufs[3 * NB:4 * NB]

        wid = lax.axis_index("s") * 2 + lax.axis_index("c")
        chunk0 = wid * NCH
        pltpu.sync_copy(idx_hbm.at[pl.ds(chunk0, NCH)], idx_v)
        pltpu.sync_copy(pe_hbm, pe_v)

        def start_gather(b, c):
            pltpu.make_async_copy(
                table_hbm.at[idx_v.at[c, 0]], gbuf[b], gsem[b]).start()

        def wait_gather(b):
            pltpu.make_async_copy(
                table_hbm.at[pl.ds(0, CH)], gbuf[b], gsem[b]).wait()

        def start_store(b, c):
            pltpu.make_async_copy(
                sbuf[b], out_hbm.at[pl.ds((chunk0 + c) * CH, CH)], ssem[b]).start()

        def wait_store(b):
            pltpu.make_async_copy(
                table_hbm.at[pl.ds(0, CH)], sbuf[b], ssem[b]).wait()

        for b in range(NB):
            start_gather(b, b)

        def outer(i, carry):
            for b in range(NB):
                c = i * NB + b
                wait_gather(b)

                @pl.when(c >= NB)
                def _():
                    wait_store(b)

                pe_base = lax.rem((chunk0 + c) * CH, L)

                @plsc.parallel_loop(0, CH, step=1, unroll=4)
                def row_body(r):
                    pos = pe_base + r
                    for j in range(VREGS):
                        sl = pl.ds(j * 16, 16)
                        sbuf[b][r, sl] = gbuf[b][r, sl] + pe_v[pos, sl]

                @pl.when(c + NB < NCH)
                def _():
                    start_gather(b, c + NB)

                start_store(b, c)
            return carry

        lax.fori_loop(0, NCH // NB, outer, 0, unroll=False)
        for b in range(NB):
            wait_store(b)

    return emb_kernel


_emb_kernel = _make_kernel()


def kernel(x, token_table):
    idx = x.reshape(BL // CH, 1, CH)
    pe = jnp.asarray(_PE_EXT)
    out = _emb_kernel(idx, token_table, pe)
    return out.reshape(B, L, D_MODEL)
